# baseline (speedup 1.0000x reference)
import functools

import jax
import jax.numpy as jnp
from jax import lax
from jax.experimental import pallas as pl
from jax.experimental.pallas import tpu as pltpu
from jax.experimental.pallas import tpu_sc as plsc

N = 10000
E = 320000
D = 128

NC = 2
NS = 16
NW = NC * NS
EPW = E // NW
RPS = 632
NP = RPS * NS
DEGW = 16

_mesh = plsc.VectorSubcoreMesh(core_axis_name="c", subcore_axis_name="s",
                               num_cores=NC, num_subcores=NS)


CD = 128
NCHUNK_DEG = 79
DEG_STG = 80


@functools.partial(
    pl.kernel,
    out_type=jax.ShapeDtypeStruct((NC * NP, DEGW), jnp.float32),
    mesh=_mesh,
    scratch_types=[
        pltpu.VMEM((DEG_STG, CD), jnp.int32),
        pltpu.VMEM((CD, DEGW), jnp.float32),
        pltpu.VMEM_SHARED((NP, DEGW), jnp.float32),
        pltpu.SemaphoreType.DMA,
    ],
)
def _deg_kernel(dstd_hbm, zeros_hbm, out_hbm, didx_all, ones_v, acc, ssem):
    c = lax.axis_index("c")
    s = lax.axis_index("s")
    w = c * NS + s

    def fill(r, _):
        ones_v[r, :] = jnp.ones((DEGW,), jnp.float32)
        return 0
    lax.fori_loop(0, CD, fill, 0)

    pltpu.sync_copy(dstd_hbm.at[w], didx_all)
    pltpu.sync_copy(zeros_hbm.at[pl.ds(0, RPS)], acc.at[pl.ds(s * RPS, RPS)])
    plsc.subcore_barrier()

    def fire(j, _):
        pltpu.async_copy(ones_v, acc.at[didx_all.at[j]], ssem, add=True)
        return 0
    lax.fori_loop(0, NCHUNK_DEG, fire, 0)

    def drain(j, _):
        pltpu.make_async_copy(ones_v, acc.at[didx_all.at[j]], ssem).wait()
        return 0
    lax.fori_loop(0, NCHUNK_DEG, drain, 0)

    plsc.subcore_barrier()
    pltpu.sync_copy(acc.at[pl.ds(s * RPS, RPS)],
                    out_hbm.at[pl.ds(c * NP + s * RPS, RPS)])


C = 128
NSLOT = 2
NCHUNK = 80
NCHUNK_USE = 80
EPWP = NCHUNK * C


@functools.partial(
    pl.kernel,
    out_type=jax.ShapeDtypeStruct((NC * NP, D), jnp.float32),
    mesh=_mesh,
    scratch_types=(
        [pltpu.VMEM((C, D), jnp.float32) for _ in range(NSLOT)]
        + [pltpu.VMEM((C,), jnp.int32) for _ in range(2 * NSLOT)]
        + [pltpu.VMEM_SHARED((NP, D), jnp.float32)]
        + [pltpu.SemaphoreType.DMA for _ in range(3 * NSLOT)]
    ),
)
def _scatter_kernel(g_hbm, srcp_hbm, dstp_hbm, zeros_hbm, out_hbm, *scr):
    rows = scr[0:NSLOT]
    sidx = scr[NSLOT:2 * NSLOT]
    didx = scr[2 * NSLOT:3 * NSLOT]
    acc = scr[3 * NSLOT]
    gsem = scr[3 * NSLOT + 1:3 * NSLOT + 1 + NSLOT]
    ssem = scr[3 * NSLOT + 1 + NSLOT:3 * NSLOT + 1 + 2 * NSLOT]
    isem = scr[3 * NSLOT + 1 + 2 * NSLOT:3 * NSLOT + 1 + 3 * NSLOT]
    c = lax.axis_index("c")
    s = lax.axis_index("s")
    w = c * NS + s
    wbase = w * EPWP

    pltpu.sync_copy(zeros_hbm.at[pl.ds(0, RPS)], acc.at[pl.ds(s * RPS, RPS)])
    plsc.subcore_barrier()

    def fire_idx(j, b):
        base = wbase + j * C
        pltpu.async_copy(srcp_hbm.at[pl.ds(base, C)], sidx[b], isem[b])
        pltpu.async_copy(dstp_hbm.at[pl.ds(base, C)], didx[b], isem[b])

    def drain_idx(j, b):
        base = wbase + j * C
        pltpu.make_async_copy(srcp_hbm.at[pl.ds(base, C)], sidx[b],
                              isem[b]).wait()
        pltpu.make_async_copy(dstp_hbm.at[pl.ds(base, C)], didx[b],
                              isem[b]).wait()

    def fire_g(b):
        pltpu.async_copy(g_hbm.at[sidx[b]], rows[b], gsem[b])

    def drain_g(b):
        pltpu.make_async_copy(g_hbm.at[sidx[b]], rows[b],
                              gsem[b]).wait()

    def fire_s(b):
        pltpu.async_copy(rows[b], acc.at[didx[b]], ssem[b], add=True)

    def drain_s(b):
        pltpu.make_async_copy(rows[b], acc.at[didx[b]],
                              ssem[b]).wait()

    for b in range(NSLOT):
        fire_idx(b, b)
    for b in range(NSLOT):
        drain_idx(b, b)
        fire_g(b)

    def body(i, _):
        j = NSLOT * i
        drain_g(0)
        fire_s(0)
        drain_g(1)
        fire_s(1)
        drain_s(0)
        fire_idx(j + 2, 0)
        drain_s(1)
        fire_idx(j + 3, 1)
        drain_idx(j + 2, 0)
        fire_g(0)
        drain_idx(j + 3, 1)
        fire_g(1)
        return 0
    lax.fori_loop(0, NCHUNK_USE // NSLOT - 1, body, 0)

    for b in range(NSLOT):
        drain_g(b)
        fire_s(b)
    for b in range(NSLOT):
        drain_s(b)

    plsc.subcore_barrier()
    pltpu.sync_copy(acc.at[pl.ds(s * RPS, RPS)],
                    out_hbm.at[pl.ds(c * NP + s * RPS, RPS)])


RB = 1000


def _tc_a_body(x_ref, w1_ref, degp_ref, g_ref, dinv_ref):
    deg = 1.0 + degp_ref[0, :, 0:1] + degp_ref[1, :, 0:1]
    dinv = lax.rsqrt(deg)
    h = jnp.dot(x_ref[...], w1_ref[...], preferred_element_type=jnp.float32)
    dinv_b = jnp.broadcast_to(dinv, (RB, D))
    g_ref[...] = h * dinv_b
    dinv_ref[...] = dinv_b


def _tc_b_body(p_ref, g1_ref, dinv_ref, b1_ref, w2_ref, g2_ref):
    dinv = dinv_ref[...]
    z = dinv * (p_ref[0] + p_ref[1] + g1_ref[...]) + b1_ref[...]
    a = jnp.maximum(z, 0.0)
    h2 = jnp.dot(a, w2_ref[...], preferred_element_type=jnp.float32)
    g2_ref[...] = h2 * dinv


def _tc_c_body(q_ref, g2_ref, dinv_ref, b2_ref, out_ref):
    out_ref[...] = (dinv_ref[...] * (q_ref[0] + q_ref[1] + g2_ref[...])
                    + b2_ref[...])


def kernel(x, edge_index, W1, b1, W2, b2):
    src = edge_index[0].astype(jnp.int32)
    dst = edge_index[1].astype(jnp.int32)
    pad = ((0, 0), (0, EPWP - EPW))
    srcp = jnp.pad(src.reshape(NW, EPW), pad).reshape(-1)
    dstp = jnp.pad(dst.reshape(NW, EPW), pad, constant_values=N).reshape(-1)
    dstd = jnp.pad(dst.reshape(NW, EPW),
                   ((0, 0), (0, DEG_STG * CD - EPW)),
                   constant_values=N).reshape(NW, DEG_STG, CD)
    zeros_wide = jnp.zeros((RPS, D), jnp.float32)
    zeros_deg = jnp.zeros((RPS, DEGW), jnp.float32)

    degp = _deg_kernel(dstd, zeros_deg).reshape(NC, NP, DEGW)[:, :N]

    grid = N // RB
    g1, dinv = pl.pallas_call(
        _tc_a_body,
        grid=(grid,),
        in_specs=[
            pl.BlockSpec((RB, D), lambda i: (i, 0)),
            pl.BlockSpec((D, D), lambda i: (0, 0)),
            pl.BlockSpec((NC, RB, DEGW), lambda i: (0, i, 0)),
        ],
        out_specs=[
            pl.BlockSpec((RB, D), lambda i: (i, 0)),
            pl.BlockSpec((RB, D), lambda i: (i, 0)),
        ],
        out_shape=[
            jax.ShapeDtypeStruct((N, D), jnp.float32),
            jax.ShapeDtypeStruct((N, D), jnp.float32),
        ],
    )(x, W1, degp)

    p = _scatter_kernel(g1, srcp, dstp, zeros_wide).reshape(NC, NP, D)[:, :N]

    g2 = pl.pallas_call(
        _tc_b_body,
        grid=(grid,),
        in_specs=[
            pl.BlockSpec((NC, RB, D), lambda i: (0, i, 0)),
            pl.BlockSpec((RB, D), lambda i: (i, 0)),
            pl.BlockSpec((RB, D), lambda i: (i, 0)),
            pl.BlockSpec((1, D), lambda i: (0, 0)),
            pl.BlockSpec((D, D), lambda i: (0, 0)),
        ],
        out_specs=pl.BlockSpec((RB, D), lambda i: (i, 0)),
        out_shape=jax.ShapeDtypeStruct((N, D), jnp.float32),
    )(p, g1, dinv, b1.reshape(1, D), W2)

    q = _scatter_kernel(g2, srcp, dstp, zeros_wide).reshape(NC, NP, D)[:, :N]

    out = pl.pallas_call(
        _tc_c_body,
        grid=(grid,),
        in_specs=[
            pl.BlockSpec((NC, RB, D), lambda i: (0, i, 0)),
            pl.BlockSpec((RB, D), lambda i: (i, 0)),
            pl.BlockSpec((RB, D), lambda i: (i, 0)),
            pl.BlockSpec((1, D), lambda i: (0, 0)),
        ],
        out_specs=pl.BlockSpec((RB, D), lambda i: (i, 0)),
        out_shape=jax.ShapeDtypeStruct((N, D), jnp.float32),
    )(q, g2, dinv, b2.reshape(1, D))

    return out


# speedup vs baseline: 1.8574x; 1.8574x over previous
import functools

import jax
import jax.numpy as jnp
from jax import lax
from jax.experimental import pallas as pl
from jax.experimental.pallas import tpu as pltpu
from jax.experimental.pallas import tpu_sc as plsc

N = 10000
E = 320000
D = 128

NC = 2
NS = 16
NW = NC * NS
EPW = E // NW
RPS = 632
NP = RPS * NS
DEGW = 16

_mesh = plsc.VectorSubcoreMesh(core_axis_name="c", subcore_axis_name="s",
                               num_cores=NC, num_subcores=NS)


CD = 128
NCHUNK_DEG = 79
DEG_STG = 80


@functools.partial(
    pl.kernel,
    out_type=jax.ShapeDtypeStruct((NC * NP, DEGW), jnp.float32),
    mesh=_mesh,
    scratch_types=[
        pltpu.VMEM((DEG_STG, CD), jnp.int32),
        pltpu.VMEM((CD, DEGW), jnp.float32),
        pltpu.VMEM_SHARED((NP, DEGW), jnp.float32),
        pltpu.SemaphoreType.DMA,
    ],
)
def _deg_kernel(dstd_hbm, zeros_hbm, out_hbm, didx_all, ones_v, acc, ssem):
    c = lax.axis_index("c")
    s = lax.axis_index("s")
    w = c * NS + s

    def fill(r, _):
        ones_v[r, :] = jnp.ones((DEGW,), jnp.float32)
        return 0
    lax.fori_loop(0, CD, fill, 0)

    pltpu.sync_copy(dstd_hbm.at[w], didx_all)
    pltpu.sync_copy(zeros_hbm.at[pl.ds(0, RPS)], acc.at[pl.ds(s * RPS, RPS)])
    plsc.subcore_barrier()

    def fire(j, _):
        pltpu.async_copy(ones_v, acc.at[didx_all.at[j]], ssem, add=True)
        return 0
    lax.fori_loop(0, NCHUNK_DEG, fire, 0)

    def drain(j, _):
        pltpu.make_async_copy(ones_v, acc.at[didx_all.at[j]], ssem).wait()
        return 0
    lax.fori_loop(0, NCHUNK_DEG, drain, 0)

    plsc.subcore_barrier()
    pltpu.sync_copy(acc.at[pl.ds(s * RPS, RPS)],
                    out_hbm.at[pl.ds(c * NP + s * RPS, RPS)])


C = 80
NSLOT = 3
NCHUNK = 126
EPWP = NCHUNK * C


@functools.partial(
    pl.kernel,
    out_type=jax.ShapeDtypeStruct((NC * NP, D), jnp.float32),
    mesh=_mesh,
    scratch_types=(
        [pltpu.VMEM((C, D), jnp.float32) for _ in range(NSLOT)]
        + [pltpu.VMEM((C,), jnp.int32) for _ in range(2 * NSLOT)]
        + [pltpu.VMEM_SHARED((NP, D), jnp.float32)]
        + [pltpu.SemaphoreType.DMA for _ in range(3 * NSLOT)]
    ),
)
def _scatter_kernel(g_hbm, srcp_hbm, dstp_hbm, zeros_hbm, out_hbm, *scr):
    rows = scr[0:NSLOT]
    sidx = scr[NSLOT:2 * NSLOT]
    didx = scr[2 * NSLOT:3 * NSLOT]
    acc = scr[3 * NSLOT]
    gsem = scr[3 * NSLOT + 1:3 * NSLOT + 1 + NSLOT]
    ssem = scr[3 * NSLOT + 1 + NSLOT:3 * NSLOT + 1 + 2 * NSLOT]
    isem = scr[3 * NSLOT + 1 + 2 * NSLOT:3 * NSLOT + 1 + 3 * NSLOT]
    c = lax.axis_index("c")
    s = lax.axis_index("s")
    w = c * NS + s
    wbase = w * EPWP

    pltpu.sync_copy(zeros_hbm.at[pl.ds(0, RPS)], acc.at[pl.ds(s * RPS, RPS)])
    plsc.subcore_barrier()

    def fire_idx(j, b):
        base = wbase + j * C
        pltpu.async_copy(srcp_hbm.at[pl.ds(base, C)], sidx[b], isem[b])
        pltpu.async_copy(dstp_hbm.at[pl.ds(base, C)], didx[b], isem[b])

    def drain_idx(j, b):
        base = wbase + j * C
        pltpu.make_async_copy(srcp_hbm.at[pl.ds(base, C)], sidx[b],
                              isem[b]).wait()
        pltpu.make_async_copy(dstp_hbm.at[pl.ds(base, C)], didx[b],
                              isem[b]).wait()

    def fire_g(b):
        pltpu.async_copy(g_hbm.at[sidx[b]], rows[b], gsem[b])

    def drain_g(b):
        pltpu.make_async_copy(g_hbm.at[sidx[b]], rows[b],
                              gsem[b]).wait()

    def fire_s(b):
        pltpu.async_copy(rows[b], acc.at[didx[b]], ssem[b], add=True)

    def drain_s(b):
        pltpu.make_async_copy(rows[b], acc.at[didx[b]],
                              ssem[b]).wait()

    for b in range(NSLOT):
        fire_idx(b, b)
    for b in range(NSLOT):
        drain_idx(b, b)
        fire_g(b)

    def body(i, _):
        j = NSLOT * i
        drain_g(0)
        fire_s(0)
        drain_g(1)
        fire_s(1)
        drain_s(0)
        fire_idx(j + 3, 0)
        drain_g(2)
        fire_s(2)
        drain_s(1)
        fire_idx(j + 4, 1)
        drain_idx(j + 3, 0)
        fire_g(0)
        drain_s(2)
        fire_idx(j + 5, 2)
        drain_idx(j + 4, 1)
        fire_g(1)
        drain_idx(j + 5, 2)
        fire_g(2)
        return 0
    lax.fori_loop(0, NCHUNK // NSLOT - 1, body, 0)

    for b in range(NSLOT):
        drain_g(b)
        fire_s(b)
    for b in range(NSLOT):
        drain_s(b)

    plsc.subcore_barrier()
    pltpu.sync_copy(acc.at[pl.ds(s * RPS, RPS)],
                    out_hbm.at[pl.ds(c * NP + s * RPS, RPS)])


RB = 1000


def _dinv(degp_ref):
    deg = 1.0 + degp_ref[0, :, 0:1] + degp_ref[1, :, 0:1]
    return jnp.broadcast_to(lax.rsqrt(deg), (RB, D))


def _tc_a_body(x_ref, w1_ref, degp_ref, g_ref):
    h = jnp.dot(x_ref[...], w1_ref[...], preferred_element_type=jnp.float32)
    g_ref[...] = h * _dinv(degp_ref)


def _tc_b_body(p_ref, g1_ref, degp_ref, b1_ref, w2_ref, g2_ref):
    dinv = _dinv(degp_ref)
    z = dinv * (p_ref[0] + p_ref[1] + g1_ref[...]) + b1_ref[...]
    a = jnp.maximum(z, 0.0)
    h2 = jnp.dot(a, w2_ref[...], preferred_element_type=jnp.float32)
    g2_ref[...] = h2 * dinv


def _tc_c_body(q_ref, g2_ref, degp_ref, b2_ref, out_ref):
    out_ref[...] = (_dinv(degp_ref) * (q_ref[0] + q_ref[1] + g2_ref[...])
                    + b2_ref[...])


def kernel(x, edge_index, W1, b1, W2, b2):
    src = edge_index[0].astype(jnp.int32)
    dst = edge_index[1].astype(jnp.int32)
    pad = ((0, 0), (0, EPWP - EPW))
    srcp = jnp.pad(src.reshape(NW, EPW), pad).reshape(-1)
    dstp = jnp.pad(dst.reshape(NW, EPW), pad, constant_values=N).reshape(-1)
    dstd = jnp.pad(dst.reshape(NW, EPW),
                   ((0, 0), (0, DEG_STG * CD - EPW)),
                   constant_values=N).reshape(NW, DEG_STG, CD)
    zeros_wide = jnp.zeros((RPS, D), jnp.float32)
    zeros_deg = jnp.zeros((RPS, DEGW), jnp.float32)

    degp = _deg_kernel(dstd, zeros_deg).reshape(NC, NP, DEGW)

    grid = N // RB
    g1 = pl.pallas_call(
        _tc_a_body,
        grid=(grid,),
        in_specs=[
            pl.BlockSpec((RB, D), lambda i: (i, 0)),
            pl.BlockSpec((D, D), lambda i: (0, 0)),
            pl.BlockSpec((NC, RB, DEGW), lambda i: (0, i, 0)),
        ],
        out_specs=pl.BlockSpec((RB, D), lambda i: (i, 0)),
        out_shape=jax.ShapeDtypeStruct((N, D), jnp.float32),
    )(x, W1, degp)

    p = _scatter_kernel(g1, srcp, dstp, zeros_wide).reshape(NC, NP, D)

    g2 = pl.pallas_call(
        _tc_b_body,
        grid=(grid,),
        in_specs=[
            pl.BlockSpec((NC, RB, D), lambda i: (0, i, 0)),
            pl.BlockSpec((RB, D), lambda i: (i, 0)),
            pl.BlockSpec((NC, RB, DEGW), lambda i: (0, i, 0)),
            pl.BlockSpec((1, D), lambda i: (0, 0)),
            pl.BlockSpec((D, D), lambda i: (0, 0)),
        ],
        out_specs=pl.BlockSpec((RB, D), lambda i: (i, 0)),
        out_shape=jax.ShapeDtypeStruct((N, D), jnp.float32),
    )(p, g1, degp, b1.reshape(1, D), W2)

    q = _scatter_kernel(g2, srcp, dstp, zeros_wide).reshape(NC, NP, D)

    out = pl.pallas_call(
        _tc_c_body,
        grid=(grid,),
        in_specs=[
            pl.BlockSpec((NC, RB, D), lambda i: (0, i, 0)),
            pl.BlockSpec((RB, D), lambda i: (i, 0)),
            pl.BlockSpec((NC, RB, DEGW), lambda i: (0, i, 0)),
            pl.BlockSpec((1, D), lambda i: (0, 0)),
        ],
        out_specs=pl.BlockSpec((RB, D), lambda i: (i, 0)),
        out_shape=jax.ShapeDtypeStruct((N, D), jnp.float32),
    )(q, g2, degp, b2.reshape(1, D))

    return out
